# baseline (speedup 1.0000x reference)
import functools
import math

import jax
import jax.numpy as jnp
from jax import lax
from jax.experimental import pallas as pl
from jax.experimental.pallas import tpu as pltpu
from jax.experimental.pallas import tpu_sc as plsc

D_MODEL = 128
LANES = 16
NUM_CORES = 2
NUM_SUBCORES = 16
NUM_WORKERS = NUM_CORES * NUM_SUBCORES
CHUNK = 64
SCALE = math.sqrt(float(D_MODEL))


@functools.lru_cache(maxsize=None)
def _build(batch: int):
    assert batch % (NUM_WORKERS * CHUNK) == 0
    bpw = batch // NUM_WORKERS
    nchunk = bpw // CHUNK

    mesh = plsc.VectorSubcoreMesh(core_axis_name="c", subcore_axis_name="s")

    @functools.partial(
        pl.kernel,
        out_type=jax.ShapeDtypeStruct((batch, D_MODEL), jnp.float32),
        mesh=mesh,
        scratch_types=[
            pltpu.VMEM((bpw,), jnp.int32),
            pltpu.VMEM((bpw, D_MODEL), jnp.float32),
            pltpu.SemaphoreType.DMA,
            [pltpu.SemaphoreType.DMA] * nchunk,
            pltpu.SemaphoreType.DMA,
        ],
    )
    def emb_kernel(x_hbm, lut_hbm, out_hbm, idx_v, rows_v, isem, gsems, wsem):
        wid = lax.axis_index("s") * NUM_CORES + lax.axis_index("c")
        base = wid * bpw

        pltpu.async_copy(x_hbm.at[pl.ds(base, bpw)], idx_v, isem).wait()

        gathers = [
            pltpu.async_copy(lut_hbm.at[idx_v.at[pl.ds(j * CHUNK, CHUNK)]],
                             rows_v.at[pl.ds(j * CHUNK, CHUNK)], gsems[j])
            for j in range(nchunk)
        ]

        writes = []
        for j in range(nchunk):
            gathers[j].wait()

            @plsc.parallel_loop(j * CHUNK, (j + 1) * CHUNK, unroll=4)
            def _(r):
                for c8 in range(D_MODEL // LANES):
                    sl = rows_v[r, pl.ds(c8 * LANES, LANES)]
                    rows_v[r, pl.ds(c8 * LANES, LANES)] = sl * SCALE

            writes.append(
                pltpu.async_copy(rows_v.at[pl.ds(j * CHUNK, CHUNK)],
                                 out_hbm.at[pl.ds(base + j * CHUNK, CHUNK)],
                                 wsem))
        for w in writes:
            w.wait()

    return emb_kernel


def kernel(x, lut):
    b0, b1 = x.shape
    batch = b0 * b1
    xf = jnp.ravel(x)
    if xf.dtype != jnp.int32:
        xf = xf.astype(jnp.int32)
    out = _build(batch)(xf, lut)
    return out.reshape(b0, b1, D_MODEL)


# speedup vs baseline: 1.0028x; 1.0028x over previous
import functools
import math

import jax
import jax.numpy as jnp
from jax import lax
from jax.experimental import pallas as pl
from jax.experimental.pallas import tpu as pltpu
from jax.experimental.pallas import tpu_sc as plsc

D_MODEL = 128
LANES = 16
NUM_CORES = 2
NUM_SUBCORES = 16
NUM_WORKERS = NUM_CORES * NUM_SUBCORES
CHUNK = 64
SCALE = math.sqrt(float(D_MODEL))


@functools.lru_cache(maxsize=None)
def _build(b0: int, b1: int):
    batch = b0 * b1
    assert batch % (NUM_WORKERS * CHUNK) == 0
    bpw = batch // NUM_WORKERS
    nchunk = bpw // CHUNK
    assert b1 % bpw == 0
    tiles_per_row = b1 // bpw

    mesh = plsc.VectorSubcoreMesh(core_axis_name="c", subcore_axis_name="s")

    @functools.partial(
        pl.kernel,
        out_type=jax.ShapeDtypeStruct((batch, D_MODEL), jnp.float32),
        mesh=mesh,
        scratch_types=[
            pltpu.VMEM((bpw,), jnp.int32),
            pltpu.VMEM((bpw, D_MODEL), jnp.float32),
            pltpu.SemaphoreType.DMA,
            [pltpu.SemaphoreType.DMA] * nchunk,
            pltpu.SemaphoreType.DMA,
        ],
    )
    def emb_kernel(x_hbm, lut_hbm, out_hbm, idx_v, rows_v, isem, gsems, wsem):
        wid = lax.axis_index("s") * NUM_CORES + lax.axis_index("c")
        base = wid * bpw
        row = wid // tiles_per_row
        col = (wid % tiles_per_row) * bpw

        pltpu.async_copy(x_hbm.at[row, pl.ds(col, bpw)], idx_v, isem).wait()

        gathers = [
            pltpu.async_copy(lut_hbm.at[idx_v.at[pl.ds(j * CHUNK, CHUNK)]],
                             rows_v.at[pl.ds(j * CHUNK, CHUNK)], gsems[j])
            for j in range(nchunk)
        ]

        writes = []
        for j in range(nchunk):
            gathers[j].wait()

            @plsc.parallel_loop(j * CHUNK, (j + 1) * CHUNK, unroll=4)
            def _(r):
                for c8 in range(D_MODEL // LANES):
                    sl = rows_v[r, pl.ds(c8 * LANES, LANES)]
                    rows_v[r, pl.ds(c8 * LANES, LANES)] = sl * SCALE

            writes.append(
                pltpu.async_copy(rows_v.at[pl.ds(j * CHUNK, CHUNK)],
                                 out_hbm.at[pl.ds(base + j * CHUNK, CHUNK)],
                                 wsem))
        for w in writes:
            w.wait()

    return emb_kernel


def kernel(x, lut):
    b0, b1 = x.shape
    if x.dtype != jnp.int32:
        x = x.astype(jnp.int32)
    out = _build(b0, b1)(x, lut)
    return out.reshape(b0, b1, D_MODEL)
